# baseline (speedup 1.0000x reference)
import functools

import jax
import jax.numpy as jnp
from jax import lax
from jax.experimental import pallas as pl
from jax.experimental.pallas import tpu as pltpu
from jax.experimental.pallas import tpu_sc as plsc

_B, _L = 4096, 50
_DW, _DF = 128, 64
_DO = _DW + _DF
_NW = 32
_BPW = _B // _NW
_GB = 2
_GT = _GB * _L
_NG = _BPW // _GB
_ND = 3

_mesh = plsc.VectorSubcoreMesh(core_axis_name="c", subcore_axis_name="s")


@functools.partial(
    pl.kernel,
    out_type=jax.ShapeDtypeStruct((_B // _GB, _GT, _DO), jnp.float32),
    mesh=_mesh,
    scratch_types=[
        [pltpu.VMEM((2, _GT), jnp.int32)] * _ND,
        [pltpu.VMEM((_GT, _DW), jnp.float32)] * _ND,
        [pltpu.VMEM((_GT, _DO), jnp.float32)] * _ND,
        [pltpu.SemaphoreType.DMA] * _ND,
        [pltpu.SemaphoreType.DMA] * _ND,
        [pltpu.SemaphoreType.DMA] * _ND,
        [pltpu.SemaphoreType.DMA] * _ND,
    ],
)
def _stack_kernel(idx_hbm, ww_hbm, wf_hbm, out_hbm,
                  cix, rf, asm, semi, semw, semf, semo):
    wid = lax.axis_index("s") * 2 + lax.axis_index("c")

    def stage_idx(c, s):
        pltpu.async_copy(idx_hbm.at[wid * _NG + c], cix[s], semi[s])

    def wait_idx(s):
        pltpu.make_async_copy(idx_hbm.at[0], cix[s], semi[s]).wait()

    def fire(c, b):
        pltpu.async_copy(ww_hbm.at[cix[b].at[0]],
                         asm[b].at[:, pl.ds(0, _DW)], semw[b])
        pltpu.async_copy(wf_hbm.at[cix[b].at[1]], rf[b], semf[b])

    def wait_write(b):
        pltpu.make_async_copy(asm[b], out_hbm.at[0], semo[b]).wait()

    def process(c, b):
        bn, bs = (b + 1) % _ND, (b + 2) % _ND
        @pl.when(c >= 2)
        def _():
            wait_write(bn)

        @pl.when(c + 2 < _NG)
        def _():
            stage_idx(c + 2, bs)

        @pl.when(c + 1 < _NG)
        def _():
            wait_idx(bn)
            fire(c + 1, bn)

        pltpu.make_async_copy(wf_hbm.at[cix[b].at[1]], rf[b], semf[b]).wait()

        @pl.loop(0, _GT, unroll=10)
        def _row(l):
            for k in range(_DF // 16):
                asm[b][l, pl.ds(_DW + 16 * k, 16)] = rf[b][l, pl.ds(16 * k, 16)]

        pltpu.make_async_copy(ww_hbm.at[cix[b].at[0]],
                              asm[b].at[:, pl.ds(0, _DW)], semw[b]).wait()
        pltpu.async_copy(asm[b], out_hbm.at[wid * _NG + c], semo[b])

    stage_idx(0, 0)
    stage_idx(1, 1)
    wait_idx(0)
    fire(0, 0)

    @pl.loop(0, _NG // _ND)
    def _trip(p):
        c0 = _ND * p
        for i in range(_ND):
            process(c0 + i, i)

    process(_NG - 1, (_NG - 1) % _ND)

    for c in (_NG - 2, _NG - 1):
        wait_write(c % _ND)


def kernel(word, feat, W_word, W_feat):
    wg = word.reshape(_B // _GB, 1, _GT).astype(jnp.int32)
    fg = feat.reshape(_B // _GB, 1, _GT).astype(jnp.int32)
    idx = jnp.concatenate([wg, fg], axis=1)
    wf_pad = jnp.pad(W_feat, ((0, 0), (0, _DW - _DF)))
    return _stack_kernel(idx, W_word, wf_pad).reshape(_B, _L, _DO)


# speedup vs baseline: 1.5548x; 1.5548x over previous
import functools

import jax
import jax.numpy as jnp
from jax import lax
from jax.experimental import pallas as pl
from jax.experimental.pallas import tpu as pltpu
from jax.experimental.pallas import tpu_sc as plsc

_B, _L = 4096, 50
_DW, _DF = 128, 64
_DO = _DW + _DF
_NW = 32
_BPW = _B // _NW
_NG = _BPW
_ND = 4

_mesh = plsc.VectorSubcoreMesh(core_axis_name="c", subcore_axis_name="s")


@functools.partial(
    pl.kernel,
    out_type=jax.ShapeDtypeStruct((_B, 1, _L, _DO), jnp.float32),
    mesh=_mesh,
    scratch_types=[
        [pltpu.VMEM((2, _L), jnp.int32)] * _ND,
        [pltpu.VMEM((_L, _DW), jnp.float32)] * _ND,
        [pltpu.VMEM((1, _L, _DO), jnp.float32)] * _ND,
        [pltpu.SemaphoreType.DMA] * _ND,
        [pltpu.SemaphoreType.DMA] * _ND,
        [pltpu.SemaphoreType.DMA] * _ND,
        [pltpu.SemaphoreType.DMA] * _ND,
    ],
)
def _stack_kernel(idx_hbm, ww_hbm, wf_hbm, out_hbm,
                  cix, rf, asm, semi, semw, semf, semo):
    wid = lax.axis_index("s") * 2 + lax.axis_index("c")

    def stage_idx(c, s):
        pltpu.async_copy(idx_hbm.at[wid * _NG + c], cix[s], semi[s])

    def wait_idx(s):
        pltpu.make_async_copy(idx_hbm.at[0], cix[s], semi[s]).wait()

    def fire(c, b):
        pltpu.async_copy(ww_hbm.at[cix[b].at[0]],
                         asm[b].at[0, :, pl.ds(0, _DW)], semw[b])
        pltpu.async_copy(wf_hbm.at[cix[b].at[1]], rf[b], semf[b])

    def wait_write(b):
        pltpu.make_async_copy(asm[b], out_hbm.at[0], semo[b]).wait()

    def process(c, b):
        bn, bs = (b + 1) % _ND, (b + 2) % _ND
        @pl.when(c >= 3)
        def _():
            wait_write(bn)

        @pl.when(c + 2 < _NG)
        def _():
            stage_idx(c + 2, bs)

        @pl.when(c + 1 < _NG)
        def _():
            wait_idx(bn)
            fire(c + 1, bn)

        pltpu.make_async_copy(wf_hbm.at[cix[b].at[1]], rf[b], semf[b]).wait()

        @pl.loop(0, _L, unroll=10)
        def _row(l):
            for k in range(_DF // 16):
                asm[b][0, l, pl.ds(_DW + 16 * k, 16)] = rf[b][l, pl.ds(16 * k, 16)]

        pltpu.make_async_copy(ww_hbm.at[cix[b].at[0]],
                              asm[b].at[0, :, pl.ds(0, _DW)], semw[b]).wait()
        pltpu.async_copy(asm[b], out_hbm.at[wid * _NG + c], semo[b])

    stage_idx(0, 0)
    stage_idx(1, 1)
    wait_idx(0)
    fire(0, 0)

    @pl.loop(0, _NG // _ND)
    def _quad(p):
        c0 = _ND * p
        for i in range(_ND):
            process(c0 + i, i)

    for c in (_NG - 3, _NG - 2, _NG - 1):
        wait_write(c % _ND)


def kernel(word, feat, W_word, W_feat):
    wg = word.reshape(_B, 1, _L).astype(jnp.int32)
    fg = feat.reshape(_B, 1, _L).astype(jnp.int32)
    idx = jnp.concatenate([wg, fg], axis=1)
    wf_pad = jnp.pad(W_feat, ((0, 0), (0, _DW - _DF)))
    return _stack_kernel(idx, W_word, wf_pad).reshape(_B, _L, _DO)
